# baseline (speedup 1.0000x reference)
import functools

import jax
import jax.numpy as jnp
from jax import lax
from jax.experimental import pallas as pl
from jax.experimental.pallas import tpu as pltpu
from jax.experimental.pallas import tpu_sc as plsc

HIDDEN = 768

_info = plsc.get_sparse_core_info()
_NC, _NS = _info.num_cores, _info.num_subcores
_NW = _NC * _NS


def _sc_gather(ids2d, word_table, chunk):
    batch, seq = ids2d.shape
    n_tokens = batch * seq
    b_per_w = n_tokens // _NW
    n_chunks = b_per_w // chunk
    w_per_batch = _NW // batch
    mesh = plsc.VectorSubcoreMesh(core_axis_name="c", subcore_axis_name="s")

    @functools.partial(
        pl.kernel,
        mesh=mesh,
        out_type=jax.ShapeDtypeStruct((n_tokens, HIDDEN), jnp.float32),
        scratch_types=[
            pltpu.VMEM((b_per_w,), jnp.int32),
            pltpu.VMEM((2, chunk, HIDDEN), jnp.float32),
            pltpu.SemaphoreType.DMA,
            pltpu.SemaphoreType.DMA,
            pltpu.SemaphoreType.DMA,
            pltpu.SemaphoreType.DMA,
        ],
    )
    def gather_kernel(idx_hbm, table_hbm, out_hbm, idx_v, rows_v,
                      gs0, gs1, ws0, ws1):
        wid = lax.axis_index("s") * _NC + lax.axis_index("c")
        b = wid // w_per_batch
        col = (wid % w_per_batch) * b_per_w
        dst_base = wid * b_per_w
        gsems = (gs0, gs1)
        wsems = (ws0, ws1)
        gcop = [None, None]
        wcop = [None, None]
        pltpu.sync_copy(idx_hbm.at[b, pl.ds(col, b_per_w)], idx_v)
        gcop[0] = pltpu.async_copy(
            table_hbm.at[idx_v.at[pl.ds(0, chunk)]], rows_v.at[0], gsems[0])
        for c in range(n_chunks):
            cur = c % 2
            nxt = (c + 1) % 2
            if c + 1 < n_chunks:
                if wcop[nxt] is not None:
                    wcop[nxt].wait()
                    wcop[nxt] = None
                gcop[nxt] = pltpu.async_copy(
                    table_hbm.at[idx_v.at[pl.ds((c + 1) * chunk, chunk)]],
                    rows_v.at[nxt], gsems[nxt])
            gcop[cur].wait()
            wcop[cur] = pltpu.async_copy(
                rows_v.at[cur],
                out_hbm.at[pl.ds(dst_base + c * chunk, chunk)], wsems[cur])
        for w in wcop:
            if w is not None:
                w.wait()

    return gather_kernel(ids2d, word_table)


def _tc_ln_body(g_ref, tt_ref, pos_ref, type_ref, w_ref, b_ref, o_ref):
    t0 = type_ref[0:1, :]
    t1 = type_ref[1:2, :]
    tt = tt_ref[...].astype(jnp.float32)
    x = g_ref[...] + pos_ref[...] + t0 + tt * (t1 - t0)
    mu = jnp.mean(x, axis=-1, keepdims=True)
    d = x - mu
    var = jnp.mean(d * d, axis=-1, keepdims=True)
    o_ref[...] = d * lax.rsqrt(var + 1e-12) * w_ref[...] + b_ref[...]


def _tc_ln(gathered, tti, pos_table, type_table, ln_w, ln_b, n_tokens, blk, seq):
    pos_blocks = seq // blk
    batch = n_tokens // seq
    grid = (pos_blocks, batch)
    tok = lambda p, b: (b * pos_blocks + p, 0)
    return pl.pallas_call(
        _tc_ln_body,
        grid=grid,
        in_specs=[
            pl.BlockSpec((blk, HIDDEN), tok),
            pl.BlockSpec((blk, 1), tok),
            pl.BlockSpec((blk, HIDDEN), lambda p, b: (p, 0)),
            pl.BlockSpec((2, HIDDEN), lambda p, b: (0, 0)),
            pl.BlockSpec((1, HIDDEN), lambda p, b: (0, 0)),
            pl.BlockSpec((1, HIDDEN), lambda p, b: (0, 0)),
        ],
        out_specs=pl.BlockSpec((blk, HIDDEN), tok),
        out_shape=jax.ShapeDtypeStruct((n_tokens, HIDDEN), jnp.float32),
    )(gathered, tti, pos_table, type_table, ln_w, ln_b)


def kernel(input_ids, token_type_ids, word_table, pos_table, type_table, ln_w, ln_b):
    batch, seq = input_ids.shape
    n_tokens = batch * seq
    tti = token_type_ids.reshape(-1, 1).astype(jnp.int32)

    gathered = _sc_gather(input_ids.astype(jnp.int32), word_table, chunk=64)
    out = _tc_ln(
        gathered, tti, pos_table, type_table,
        ln_w.reshape(1, HIDDEN), ln_b.reshape(1, HIDDEN),
        n_tokens, blk=2048, seq=seq,
    )
    return out.reshape(batch, seq, HIDDEN)


# speedup vs baseline: 1.0056x; 1.0056x over previous
import functools

import jax
import jax.numpy as jnp
from jax import lax
from jax.experimental import pallas as pl
from jax.experimental.pallas import tpu as pltpu
from jax.experimental.pallas import tpu_sc as plsc

HIDDEN = 768

_info = plsc.get_sparse_core_info()
_NC, _NS = _info.num_cores, _info.num_subcores
_NW = _NC * _NS


def _sc_gather(ids2d, word_table, chunk):
    batch, seq = ids2d.shape
    n_tokens = batch * seq
    b_per_w = n_tokens // _NW
    n_chunks = b_per_w // chunk
    w_per_batch = _NW // batch
    mesh = plsc.VectorSubcoreMesh(core_axis_name="c", subcore_axis_name="s")

    @functools.partial(
        pl.kernel,
        mesh=mesh,
        out_type=jax.ShapeDtypeStruct((n_tokens, HIDDEN), jnp.float32),
        scratch_types=[
            pltpu.VMEM((2, chunk), jnp.int32),
            pltpu.VMEM((2, chunk, HIDDEN), jnp.float32),
            pltpu.SemaphoreType.DMA,
            pltpu.SemaphoreType.DMA,
        ],
    )
    def gather_kernel(idx_hbm, table_hbm, out_hbm, idx_v, rows_v, sem0, sem1):
        wid = lax.axis_index("s") * _NC + lax.axis_index("c")
        b = wid // w_per_batch
        col = (wid % w_per_batch) * b_per_w
        dst_base = wid * b_per_w
        sems = (sem0, sem1)
        copies = [None, None]
        pltpu.sync_copy(idx_hbm.at[b, pl.ds(col, chunk)], idx_v.at[0])
        copies[0] = pltpu.async_copy(table_hbm.at[idx_v.at[0]], rows_v.at[0], sems[0])
        for c in range(n_chunks):
            cur = c % 2
            nxt = (c + 1) % 2
            if c + 1 < n_chunks:
                pltpu.sync_copy(idx_hbm.at[b, pl.ds(col + (c + 1) * chunk, chunk)],
                                idx_v.at[nxt])
                copies[nxt] = pltpu.async_copy(
                    table_hbm.at[idx_v.at[nxt]], rows_v.at[nxt], sems[nxt])
            copies[cur].wait()
            pltpu.sync_copy(rows_v.at[cur],
                            out_hbm.at[pl.ds(dst_base + c * chunk, chunk)])

    return gather_kernel(ids2d, word_table)


def _tc_ln_body(g_ref, tt_ref, pos_ref, type_ref, w_ref, b_ref, o_ref):
    t0 = type_ref[0:1, :]
    t1 = type_ref[1:2, :]
    tt = tt_ref[...].astype(jnp.float32)
    x = g_ref[...] + pos_ref[...] + t0 + tt * (t1 - t0)
    mu = jnp.mean(x, axis=-1, keepdims=True)
    d = x - mu
    var = jnp.mean(d * d, axis=-1, keepdims=True)
    o_ref[...] = d * lax.rsqrt(var + 1e-12) * w_ref[...] + b_ref[...]


def _tc_ln(gathered, tti, pos_table, type_table, ln_w, ln_b, n_tokens, blk, seq):
    pos_blocks = seq // blk
    batch = n_tokens // seq
    grid = (pos_blocks, batch)
    tok = lambda p, b: (b * pos_blocks + p, 0)
    return pl.pallas_call(
        _tc_ln_body,
        grid=grid,
        in_specs=[
            pl.BlockSpec((blk, HIDDEN), tok),
            pl.BlockSpec((blk, 1), tok),
            pl.BlockSpec((blk, HIDDEN), lambda p, b: (p, 0)),
            pl.BlockSpec((2, HIDDEN), lambda p, b: (0, 0)),
            pl.BlockSpec((1, HIDDEN), lambda p, b: (0, 0)),
            pl.BlockSpec((1, HIDDEN), lambda p, b: (0, 0)),
        ],
        out_specs=pl.BlockSpec((blk, HIDDEN), tok),
        out_shape=jax.ShapeDtypeStruct((n_tokens, HIDDEN), jnp.float32),
    )(gathered, tti, pos_table, type_table, ln_w, ln_b)


def kernel(input_ids, token_type_ids, word_table, pos_table, type_table, ln_w, ln_b):
    batch, seq = input_ids.shape
    n_tokens = batch * seq
    tti = token_type_ids.reshape(-1, 1).astype(jnp.int32)

    gathered = _sc_gather(input_ids.astype(jnp.int32), word_table, chunk=64)
    out = _tc_ln(
        gathered, tti, pos_table, type_table,
        ln_w.reshape(1, HIDDEN), ln_b.reshape(1, HIDDEN),
        n_tokens, blk=2048, seq=seq,
    )
    return out.reshape(batch, seq, HIDDEN)


# speedup vs baseline: 1.0075x; 1.0019x over previous
import functools

import jax
import jax.numpy as jnp
from jax import lax
from jax.experimental import pallas as pl
from jax.experimental.pallas import tpu as pltpu
from jax.experimental.pallas import tpu_sc as plsc

HIDDEN = 768

_info = plsc.get_sparse_core_info()
_NC, _NS = _info.num_cores, _info.num_subcores
_NW = _NC * _NS


def _sc_gather(ids2d, word_table, chunk):
    batch, seq = ids2d.shape
    n_tokens = batch * seq
    b_per_w = n_tokens // _NW
    n_chunks = b_per_w // chunk
    w_per_batch = _NW // batch
    mesh = plsc.VectorSubcoreMesh(core_axis_name="c", subcore_axis_name="s")

    @functools.partial(
        pl.kernel,
        mesh=mesh,
        out_type=jax.ShapeDtypeStruct((n_tokens, HIDDEN), jnp.float32),
        scratch_types=[
            pltpu.VMEM((2, chunk), jnp.int32),
            pltpu.VMEM((2, chunk, HIDDEN), jnp.float32),
            pltpu.SemaphoreType.DMA,
            pltpu.SemaphoreType.DMA,
        ],
    )
    def gather_kernel(idx_hbm, table_hbm, out_hbm, idx_v, rows_v, sem0, sem1):
        wid = lax.axis_index("s") * _NC + lax.axis_index("c")
        b = wid // w_per_batch
        col = (wid % w_per_batch) * b_per_w
        dst_base = wid * b_per_w
        sems = (sem0, sem1)
        copies = [None, None]
        pltpu.sync_copy(idx_hbm.at[b, pl.ds(col, chunk)], idx_v.at[0])
        copies[0] = pltpu.async_copy(table_hbm.at[idx_v.at[0]], rows_v.at[0], sems[0])
        for c in range(n_chunks):
            cur = c % 2
            nxt = (c + 1) % 2
            if c + 1 < n_chunks:
                pltpu.sync_copy(idx_hbm.at[b, pl.ds(col + (c + 1) * chunk, chunk)],
                                idx_v.at[nxt])
                copies[nxt] = pltpu.async_copy(
                    table_hbm.at[idx_v.at[nxt]], rows_v.at[nxt], sems[nxt])
            copies[cur].wait()
            pltpu.sync_copy(rows_v.at[cur],
                            out_hbm.at[pl.ds(dst_base + c * chunk, chunk)])

    return gather_kernel(ids2d, word_table)


def _tc_ln_body(g_ref, tt_ref, pos_ref, type_ref, w_ref, b_ref, o_ref):
    t0 = type_ref[0:1, :]
    t1 = type_ref[1:2, :]
    tt = tt_ref[...].astype(jnp.float32)
    x = g_ref[...] + pos_ref[...] + t0 + tt * (t1 - t0)
    mu = jnp.mean(x, axis=-1, keepdims=True)
    ex2 = jnp.mean(x * x, axis=-1, keepdims=True)
    var = ex2 - mu * mu
    r = lax.rsqrt(var + 1e-12)
    o_ref[...] = (x - mu) * (r * w_ref[...]) + b_ref[...]


def _tc_ln(gathered, tti, pos_table, type_table, ln_w, ln_b, n_tokens, blk, seq):
    pos_blocks = seq // blk
    batch = n_tokens // seq
    grid = (pos_blocks, batch)
    tok = lambda p, b: (b * pos_blocks + p, 0)
    return pl.pallas_call(
        _tc_ln_body,
        grid=grid,
        in_specs=[
            pl.BlockSpec((blk, HIDDEN), tok),
            pl.BlockSpec((blk, 1), tok),
            pl.BlockSpec((blk, HIDDEN), lambda p, b: (p, 0)),
            pl.BlockSpec((2, HIDDEN), lambda p, b: (0, 0)),
            pl.BlockSpec((1, HIDDEN), lambda p, b: (0, 0)),
            pl.BlockSpec((1, HIDDEN), lambda p, b: (0, 0)),
        ],
        out_specs=pl.BlockSpec((blk, HIDDEN), tok),
        out_shape=jax.ShapeDtypeStruct((n_tokens, HIDDEN), jnp.float32),
    )(gathered, tti, pos_table, type_table, ln_w, ln_b)


def kernel(input_ids, token_type_ids, word_table, pos_table, type_table, ln_w, ln_b):
    batch, seq = input_ids.shape
    n_tokens = batch * seq
    tti = token_type_ids.reshape(-1, 1).astype(jnp.int32)

    gathered = _sc_gather(input_ids.astype(jnp.int32), word_table, chunk=64)
    out = _tc_ln(
        gathered, tti, pos_table, type_table,
        ln_w.reshape(1, HIDDEN), ln_b.reshape(1, HIDDEN),
        n_tokens, blk=2048, seq=seq,
    )
    return out.reshape(batch, seq, HIDDEN)
